# Initial kernel scaffold; baseline (speedup 1.0000x reference)
import jax
import jax.numpy as jnp
from jax.experimental import pallas as pl


def kernel(features, points0, points1, points2, points3, neighbors0, neighbors1, neighbors2, neighbors3, sub_neighbors0, sub_neighbors1, sub_neighbors2, up_neighbors1, up_neighbors2, params):
    raise NotImplementedError("write your pallas kernel here")


# speedup vs baseline: 1.1834x; 1.1834x over previous
import functools

import jax
import jax.numpy as jnp
import numpy as np
from jax import lax
from jax.experimental import pallas as pl
from jax.experimental.pallas import tpu as pltpu
from jax.experimental.pallas import tpu_sc as plsc

KER = 15
R0 = 0.0625
S0 = 0.05
GROUPS = 32
K_NEIGH = 32
KP_PTS = np.random.RandomState(42).uniform(-1.0, 1.0, (KER, 3)).astype(np.float32)
NW = 32


def _rup(x, m):
    return (x + m - 1) // m * m


def _lrelu(x):
    return jnp.where(x >= 0, x, 0.1 * x)


def _bb(x):
    return x.astype(jnp.bfloat16).astype(jnp.float32)


def _dot_ref(a, b):
    return jnp.dot(a.astype(jnp.bfloat16), b.astype(jnp.bfloat16),
                   preferred_element_type=jnp.float32)


def _gn_rows(y, g, b):
    n, c = y.shape
    gs = c // GROUPS
    yr = y.reshape(n, GROUPS, gs)
    m = jnp.mean(yr, axis=2, keepdims=True)
    v = jnp.mean((yr - m) ** 2, axis=2, keepdims=True)
    yr = (yr - m) / jnp.sqrt(v + 1e-5)
    return yr.reshape(n, c) * g + b


def _sc_gather(table, idx, d):
    b = idx.shape[0]
    chunk = 128 if d <= 512 else 64
    assert b % (NW * chunk) == 0, (b, d, chunk)
    per_w = b // NW
    mesh = plsc.VectorSubcoreMesh(core_axis_name="c", subcore_axis_name="s")

    @functools.partial(
        pl.kernel,
        mesh=mesh,
        out_type=jax.ShapeDtypeStruct((b, d), jnp.float32),
        scratch_types=[
            pltpu.VMEM((chunk,), jnp.int32),
            pltpu.VMEM((chunk, d), jnp.float32),
            pltpu.SemaphoreType.DMA,
        ],
        compiler_params=pltpu.CompilerParams(use_tc_tiling_on_sc=False),
    )
    def gk(table_hbm, idx_hbm, out_hbm, idx_v, rows_v, sem):
        wid = lax.axis_index("s") * 2 + lax.axis_index("c")
        base = wid * per_w

        def body(g, carry):
            off = base + g * chunk
            pltpu.sync_copy(idx_hbm.at[pl.ds(off, chunk)], idx_v)
            pltpu.async_copy(table_hbm.at[idx_v], rows_v, sem).wait()
            pltpu.sync_copy(rows_v, out_hbm.at[pl.ds(off, chunk)])
            return carry

        lax.fori_loop(0, per_w // chunk, body, 0)

    return gk(table, idx)


def _pad_idx(idx, b_pad):
    flat = idx.reshape(-1).astype(jnp.int32)
    return jnp.pad(flat, (0, b_pad - flat.shape[0]))


def _full(shape):
    return pl.BlockSpec(shape, lambda i: tuple(0 for _ in shape))


def _rows(bn, c):
    return pl.BlockSpec((bn, c), lambda i: (i, 0))


def _linear_gn_lrelu(x, w, g, b, bn=256):
    npad, cin = x.shape
    cout = w.shape[1]

    def body(x_ref, w_ref, g_ref, b_ref, o_ref):
        y = _dot_ref(x_ref[:], w_ref[:])
        o_ref[:] = _lrelu(_gn_rows(y, g_ref[:], b_ref[:]))

    return pl.pallas_call(
        body,
        grid=(npad // bn,),
        in_specs=[_rows(bn, cin), _full((cin, cout)), _full((1, cout)), _full((1, cout))],
        out_specs=_rows(bn, cout),
        out_shape=jax.ShapeDtypeStruct((npad, cout), jnp.float32),
    )(x, w, g, b)


def _influence(qp, posg, kpt, sigma, bm):
    rel = posg.reshape(bm, K_NEIGH, 16) - qp.reshape(bm, 1, 16)
    rel2 = rel.reshape(bm * K_NEIGH, 16)
    lane = lax.broadcasted_iota(jnp.int32, (1, 16), 1)
    m3 = lane < 3
    cols = []
    for p_ in range(KER):
        dif = rel2 - kpt[p_ : p_ + 1, :]
        d2 = jnp.sum(jnp.where(m3, dif * dif, 0.0), axis=1, keepdims=True)
        dist = jnp.sqrt(d2 + 1e-12)
        cols.append(jnp.maximum(0.0, 1.0 - dist / sigma))
    infl = jnp.concatenate(cols, axis=1)
    return rel2, infl


def _conv0(qp, posg, w0, g0, b0, bm=128):
    npad = qp.shape[0]
    kpt = jnp.pad(KP_PTS * R0, ((0, 0), (0, 13)))

    def body(qp_ref, posg_ref, kpt_ref, w_ref, g_ref, b_ref, o_ref):
        rel2, infl = _influence(qp_ref[:], posg_ref[:], kpt_ref[:], S0, bm)
        lane = lax.broadcasted_iota(jnp.int32, (1, 16), 1)
        feat = jnp.sum(jnp.where(lane == 3, rel2, 0.0), axis=1, keepdims=True)
        feat3 = _bb(feat).reshape(bm, K_NEIGH, 1)
        infl3 = _bb(infl).reshape(bm, K_NEIGH, KER)
        s = jnp.concatenate(
            [jnp.sum(infl3[:, :, p : p + 1] * feat3, axis=1) for p in range(KER)], axis=1
        )
        y = _dot_ref(s, w_ref[:])
        o_ref[:] = _lrelu(_gn_rows(y, g_ref[:], b_ref[:]))

    return pl.pallas_call(
        body,
        grid=(npad // bm,),
        in_specs=[
            _rows(bm, 16),
            _rows(bm * K_NEIGH, 16),
            _full((KER, 16)),
            _full((KER, 64)),
            _full((1, 64)),
            _full((1, 64)),
        ],
        out_specs=_rows(bm, 64),
        out_shape=jax.ShapeDtypeStruct((npad, 64), jnp.float32),
    )(qp, posg, kpt, w0, g0, b0)


def _res_tail(qp, posg, nf, sc, p, radius, sigma, mode, bm=128):
    npad = qp.shape[0]
    c = nf.shape[1]
    cout = p["w3"].shape[1]
    cin = sc.shape[1]
    kpt = jnp.pad(KP_PTS * radius, ((0, 0), (0, 13)))
    kwf = p["kw"].reshape(KER * c, c)

    def body(*refs):
        if mode == "ws":
            (qp_ref, posg_ref, nf_ref, sc_ref, kpt_ref, kw_ref,
             g2_ref, b2_ref, w3_ref, g3_ref, b3_ref, ws_ref, gs_ref, bs_ref, o_ref) = refs
        else:
            (qp_ref, posg_ref, nf_ref, sc_ref, kpt_ref, kw_ref,
             g2_ref, b2_ref, w3_ref, g3_ref, b3_ref, o_ref) = refs
        _, infl = _influence(qp_ref[:], posg_ref[:], kpt_ref[:], sigma, bm)
        infl3 = _bb(infl).reshape(bm, K_NEIGH, KER)
        nf3 = _bb(nf_ref[:]).reshape(bm, K_NEIGH, c)
        s = jnp.concatenate(
            [jnp.sum(infl3[:, :, p_ : p_ + 1] * nf3, axis=1) for p_ in range(KER)], axis=1
        )
        x = _dot_ref(s, kw_ref[:])
        x = _lrelu(_gn_rows(x, g2_ref[:], b2_ref[:]))
        x = _dot_ref(x, w3_ref[:])
        x = _gn_rows(x, g3_ref[:], b3_ref[:])
        if mode == "ws":
            sh = _gn_rows(_dot_ref(sc_ref[:], ws_ref[:]), gs_ref[:], bs_ref[:])
        elif mode == "id":
            sh = sc_ref[:]
        else:
            sh = jnp.max(sc_ref[:].reshape(bm, K_NEIGH, cin), axis=1)
        o_ref[:] = _lrelu(x + sh)

    sc_spec = _rows(bm * K_NEIGH, cin) if mode == "maxg" else _rows(bm, cin)
    in_specs = [
        _rows(bm, 16),
        _rows(bm * K_NEIGH, 16),
        _rows(bm * K_NEIGH, c),
        sc_spec,
        _full((KER, 16)),
        _full((KER * c, c)),
        _full((1, c)),
        _full((1, c)),
        _full((c, cout)),
        _full((1, cout)),
        _full((1, cout)),
    ]
    args = [qp, posg, nf, sc, kpt, kwf,
            p["g2x"], p["b2x"], p["w3"], p["g3x"], p["b3x"]]
    if mode == "ws":
        in_specs += [_full((cin, cout)), _full((1, cout)), _full((1, cout))]
        args += [p["ws"], p["gsx"], p["bsx"]]
    return pl.pallas_call(
        body,
        grid=(npad // bm,),
        in_specs=in_specs,
        out_specs=_rows(bm, cout),
        out_shape=jax.ShapeDtypeStruct((npad, cout), jnp.float32),
    )(*args)


def _collapsed_tail(sc, p, mode, bm=256):
    npad = sc.shape[0] if mode == "ws" else sc.shape[0] // K_NEIGH
    cin = sc.shape[1]
    mid = p["w3"].shape[0]
    cout = p["w3"].shape[1]

    def body(*refs):
        if mode == "ws":
            sc_ref, b2_ref, w3_ref, g3_ref, b3_ref, ws_ref, gs_ref, bs_ref, o_ref = refs
        else:
            sc_ref, b2_ref, w3_ref, g3_ref, b3_ref, o_ref = refs
        xrow = _gn_rows(_dot_ref(_lrelu(b2_ref[:]), w3_ref[:]),
                        g3_ref[:], b3_ref[:])
        if mode == "ws":
            sh = _gn_rows(_dot_ref(sc_ref[:], ws_ref[:]), gs_ref[:], bs_ref[:])
        else:
            sh = jnp.max(sc_ref[:].reshape(bm, K_NEIGH, cin), axis=1)
        o_ref[:] = _lrelu(xrow + sh)

    sc_spec = _rows(bm * K_NEIGH, cin) if mode == "maxg" else _rows(bm, cin)
    in_specs = [sc_spec, _full((1, mid)), _full((mid, cout)), _full((1, cout)), _full((1, cout))]
    args = [sc, p["b2x"], p["w3"], p["g3x"], p["b3x"]]
    if mode == "ws":
        in_specs += [_full((cin, cout)), _full((1, cout)), _full((1, cout))]
        args += [p["ws"], p["gsx"], p["bsx"]]
    return pl.pallas_call(
        body,
        grid=(npad // bm,),
        in_specs=in_specs,
        out_specs=_rows(bm, cout),
        out_shape=jax.ShapeDtypeStruct((npad, cout), jnp.float32),
    )(*args)


def _dec(a, x, wa, wx, g, b, act, npad, bn=256):
    ca, cx = a.shape[1], x.shape[1]
    cout = wa.shape[1]

    def body(*refs):
        if act:
            a_ref, x_ref, wa_ref, wx_ref, g_ref, b_ref, o_ref = refs
        else:
            a_ref, x_ref, wa_ref, wx_ref, o_ref = refs
        y = _dot_ref(a_ref[:], wa_ref[:]) + _dot_ref(x_ref[:], wx_ref[:])
        if act:
            y = _lrelu(_gn_rows(y, g_ref[:], b_ref[:]))
        o_ref[:] = y

    in_specs = [_rows(bn, ca), _rows(bn, cx), _full((ca, cout)), _full((cx, cout))]
    args = [a, x, wa, wx]
    if act:
        in_specs += [_full((1, cout)), _full((1, cout))]
        args += [g, b]
    return pl.pallas_call(
        body,
        grid=(npad // bn,),
        in_specs=in_specs,
        out_specs=_rows(bn, cout),
        out_shape=jax.ShapeDtypeStruct((npad, cout), jnp.float32),
    )(*args)


def _prep_p(p):
    q = dict(p)
    for k in ("g1", "b1", "g2", "b2", "g3", "b3", "gs", "bs"):
        if k in p:
            q[k + "x"] = p[k][None, :]
    return q


def kernel(features, points0, points1, points2, points3, neighbors0, neighbors1,
           neighbors2, neighbors3, sub_neighbors0, sub_neighbors1, sub_neighbors2,
           up_neighbors1, up_neighbors2, params):
    n0, n1, n2, n3 = points0.shape[0], points1.shape[0], points2.shape[0], points3.shape[0]
    p0, p1, p2, p3 = _rup(n0, 256), _rup(n1, 256), _rup(n2, 256), _rup(n3, 256)

    def padpts(pts, npad, feat=None):
        cols = [pts] if feat is None else [pts, feat]
        t = jnp.concatenate(cols + [jnp.zeros((pts.shape[0], 16 - sum(c.shape[1] for c in cols)), jnp.float32)], axis=1)
        return jnp.pad(t, ((0, npad - pts.shape[0]), (0, 0)))

    t0 = padpts(points0, n0, features)[:n0]
    t1 = padpts(points1, n1)[:n1]
    t2 = padpts(points2, n2)[:n2]
    t3 = padpts(points3, n3)[:n3]
    qp0 = padpts(points0, p0)
    qp1 = padpts(points1, p1)
    qp2 = padpts(points2, p2)
    qp3 = padpts(points3, p3)

    rb = [_prep_p(p) for p in params["res"]]

    pg_n0 = _sc_gather(t0, _pad_idx(neighbors0, p0 * K_NEIGH), 16)
    pg_n1 = _sc_gather(t1, _pad_idx(neighbors1, p1 * K_NEIGH), 16)
    pg_s1 = _sc_gather(t1, _pad_idx(sub_neighbors1, p2 * K_NEIGH), 16)
    pg_n2 = _sc_gather(t2, _pad_idx(neighbors2, p2 * K_NEIGH), 16)
    pg_s2 = _sc_gather(t2, _pad_idx(sub_neighbors2, p3 * K_NEIGH), 16)
    pg_n3 = _sc_gather(t3, _pad_idx(neighbors3, p3 * K_NEIGH), 16)

    w0 = params["conv0_w"].reshape(KER, 64)
    f0 = _conv0(qp0, pg_n0, w0, params["conv0_g"][None, :], params["conv0_b"][None, :])

    f1 = _collapsed_tail(f0, rb[0], "ws")

    g_sc1 = _sc_gather(f1, _pad_idx(sub_neighbors0, p1 * K_NEIGH), 128)
    f2 = _collapsed_tail(g_sc1, rb[1], "maxg")

    idx_n1 = _pad_idx(neighbors1, p1 * K_NEIGH)
    idx_s1 = _pad_idx(sub_neighbors1, p2 * K_NEIGH)
    idx_n2 = _pad_idx(neighbors2, p2 * K_NEIGH)
    idx_s2 = _pad_idx(sub_neighbors2, p3 * K_NEIGH)
    idx_n3 = _pad_idx(neighbors3, p3 * K_NEIGH)

    x1 = _linear_gn_lrelu(f2, rb[2]["w1"], rb[2]["g1x"], rb[2]["b1x"])
    nf = _sc_gather(x1, idx_n1, 64)
    f3 = _res_tail(qp1, pg_n1, nf, f2, rb[2], R0 * 2, S0 * 2, "ws")

    x1 = _linear_gn_lrelu(f3, rb[3]["w1"], rb[3]["g1x"], rb[3]["b1x"])
    nf = _sc_gather(x1, idx_n1, 64)
    f4 = _res_tail(qp1, pg_n1, nf, f3, rb[3], R0 * 2, S0 * 2, "id")

    x1 = _linear_gn_lrelu(f4, rb[4]["w1"], rb[4]["g1x"], rb[4]["b1x"])
    nf = _sc_gather(x1, idx_s1, 64)
    gmax = _sc_gather(f4, idx_s1, 256)
    f5 = _res_tail(qp2, pg_s1, nf, gmax, rb[4], R0 * 2, S0 * 2, "maxg")

    x1 = _linear_gn_lrelu(f5, rb[5]["w1"], rb[5]["g1x"], rb[5]["b1x"])
    nf = _sc_gather(x1, idx_n2, 128)
    f6 = _res_tail(qp2, pg_n2, nf, f5, rb[5], R0 * 4, S0 * 4, "ws")

    x1 = _linear_gn_lrelu(f6, rb[6]["w1"], rb[6]["g1x"], rb[6]["b1x"])
    nf = _sc_gather(x1, idx_n2, 128)
    f7 = _res_tail(qp2, pg_n2, nf, f6, rb[6], R0 * 4, S0 * 4, "id")

    x1 = _linear_gn_lrelu(f7, rb[7]["w1"], rb[7]["g1x"], rb[7]["b1x"])
    nf = _sc_gather(x1, idx_s2, 128)
    gmax = _sc_gather(f7, idx_s2, 512)
    f8 = _res_tail(qp3, pg_s2, nf, gmax, rb[7], R0 * 4, S0 * 4, "maxg")

    x1 = _linear_gn_lrelu(f8, rb[8]["w1"], rb[8]["g1x"], rb[8]["b1x"])
    nf = _sc_gather(x1, idx_n3, 256)
    f9 = _res_tail(qp3, pg_n3, nf, f8, rb[8], R0 * 8, S0 * 8, "ws")

    x1 = _linear_gn_lrelu(f9, rb[9]["w1"], rb[9]["g1x"], rb[9]["b1x"])
    nf = _sc_gather(x1, idx_n3, 256)
    f10 = _res_tail(qp3, pg_n3, nf, f9, rb[9], R0 * 8, S0 * 8, "id")

    bpad_d1 = _rup(p2, NW * 64)
    gd1 = _sc_gather(f10, _pad_idx(up_neighbors2[:, 0], bpad_d1), 1024)
    lat2 = _dec(gd1[:p2], f7, params["dec1_w"][:1024], params["dec1_w"][1024:],
                params["dec1_g"][None, :], params["dec1_b"][None, :], True, p2)

    bpad_d2 = _rup(p1, NW * 128)
    gd2 = _sc_gather(lat2, _pad_idx(up_neighbors1[:, 0], bpad_d2), 512)
    out = _dec(gd2[:p1], f4, params["last_w"][:512], params["last_w"][512:],
               None, None, False, p1)
    return out[:n1]
